# baseline (speedup 1.0000x reference)
import functools

import jax
import jax.numpy as jnp
from jax import lax
from jax.experimental import pallas as pl
from jax.experimental.pallas import tpu as pltpu
from jax.experimental.pallas import tpu_sc as plsc

_M = 100000
_D = 512
_B = 16384
_C = 1000
_CP = 1024

_NC = 2
_NS = 16
_NW = _NC * _NS
_WCOPY = 25
_R = 4000
_BPW = _B // _NW
_CHUNK = 96
_LISTPAD = _R + _CHUNK + 16

def _i16():
    return lax.broadcasted_iota(jnp.int32, (16,), 0)


def _take16(vec, ind):
    return lax.gather(
        vec, ind[:, None],
        lax.GatherDimensionNumbers(offset_dims=(), collapsed_slice_dims=(0,),
                                   start_index_map=(0,)),
        (1,), mode=lax.GatherScatterMode.PROMISE_IN_BOUNDS)


def _sc_update_kernel(bx, by, bu, x, y, idx, unc,
                      nbx, nby, nbu, hist,
                      idx_v, y_v, u_v, by_v, bu_v, win_v,
                      wl_v, rl_v, wl96, rl96, rows_v, lab_v, hist_v,
                      sem_copy, sem_io):
    wid = lax.axis_index("s") * _NC + lax.axis_index("c")
    lo = wid * _R

    is_range = wid < _WCOPY

    copy_dma = pltpu.make_async_copy(
        bx.at[pl.ds(lo * jnp.where(is_range, 1, 0), _R)],
        nbx.at[pl.ds(lo * jnp.where(is_range, 1, 0), _R)],
        sem_copy)

    @pl.when(is_range)
    def _():
        copy_dma.start()

    pltpu.sync_copy(idx, idx_v)
    pltpu.sync_copy(y, y_v)
    pltpu.sync_copy(unc, u_v)

    def _hist_zero(k, _):
        hist_v[pl.ds(k * 16, 16)] = jnp.zeros((16,), jnp.int32)
        return 0
    lax.fori_loop(0, _CP // 16, _hist_zero, 0, unroll=4)

    ibase = wid * _BPW
    nchunks = _BPW // 128
    for c in range(nchunks):
        off = ibase + c * 128
        pltpu.sync_copy(by.at[idx_v.at[pl.ds(off, 128)]], lab_v)

        def _hist_acc(k, _):
            mall = jnp.full((16,), True)
            lab = lab_v[pl.ds(k * 16, 16)]
            plsc.addupdate_scatter(hist_v, [lab],
                                   jnp.full((16,), -1, jnp.int32), mask=mall)
            yy = y_v[pl.ds(off + k * 16, 16)]
            plsc.addupdate_scatter(hist_v, [yy],
                                   jnp.ones((16,), jnp.int32), mask=mall)
            return 0
        lax.fori_loop(0, 8, _hist_acc, 0, unroll=2)

    pltpu.sync_copy(hist_v, hist.at[wid])

    @pl.when(is_range)
    def _():
        def _win_init(k, _):
            win_v[pl.ds(k * 16, 16)] = jnp.full((16,), -1, jnp.int32)
            return 0
        lax.fori_loop(0, _R // 16, _win_init, 0, unroll=4)

        shift_up = jnp.minimum(_i16() + 1, 15)

        def _win_build(k, _):
            v = idx_v[pl.ds(k * 16, 16)]
            rel = v - lo
            valid = (rel >= 0) & (rel < _R)
            ival = k * 16 + _i16()
            comp = jnp.where(valid, rel * _B + ival, jnp.int32(0x7FFFFFFF))
            scomp = lax.sort(comp)
            nxt = _take16(scomp, shift_up)
            srel = lax.shift_right_logical(scomp, 14)
            keep = ((srel != lax.shift_right_logical(nxt, 14)) | (_i16() == 15))
            keep = keep & (scomp != jnp.int32(0x7FFFFFFF))
            srel_c = jnp.where(keep, srel, 0)
            sival = jnp.where(keep, scomp & (_B - 1), 0)
            plsc.store_scatter(win_v, [srel_c], sival, mask=keep)
            return 0
        lax.fori_loop(0, _B // 16, _win_build, 0, unroll=2)

        pltpu.sync_copy(by.at[pl.ds(lo, _R)], by_v)
        pltpu.sync_copy(bu.at[pl.ds(lo, _R)], bu_v)

        def _merge(k, off):
            w = win_v[pl.ds(k * 16, 16)]
            m = w >= 0
            ws = jnp.where(m, w, 0)
            yw = plsc.load_gather(y_v, [ws], mask=m)
            uw = plsc.load_gather(u_v, [ws], mask=m)
            by_v[pl.ds(k * 16, 16)] = jnp.where(m, yw, by_v[pl.ds(k * 16, 16)])
            bu_v[pl.ds(k * 16, 16)] = jnp.where(m, uw, bu_v[pl.ds(k * 16, 16)])
            plsc.store_compressed(wl_v.at[pl.ds(off, 16)], ws, mask=m)
            plsc.store_compressed(rl_v.at[pl.ds(off, 16)],
                                  lo + k * 16 + _i16(), mask=m)
            return off + jnp.sum(m.astype(jnp.int32))
        nw = lax.fori_loop(0, _R // 16, _merge, jnp.int32(0))

        pltpu.sync_copy(by_v, nby.at[pl.ds(lo, _R)])
        pltpu.sync_copy(bu_v, nbu.at[pl.ds(lo, _R)])

        @pl.when(nw > 0)
        def _():
            lastw = wl_v[pl.ds(nw - 1, 16)]
            lastr = rl_v[pl.ds(nw - 1, 16)]
            z16 = jnp.zeros((16,), jnp.int32)
            padw = _take16(lastw, z16)
            padr = _take16(lastr, z16)
            for t in range(_CHUNK // 16):
                wl_v[pl.ds(nw + t * 16, 16)] = padw
                rl_v[pl.ds(nw + t * 16, 16)] = padr

        copy_dma.wait()

        nc = (nw + _CHUNK - 1) // _CHUNK

        def _scatter(c, _):
            base = c * _CHUNK
            for t in range(_CHUNK // 16):
                wl96[pl.ds(t * 16, 16)] = wl_v[pl.ds(base + t * 16, 16)]
                rl96[pl.ds(t * 16, 16)] = rl_v[pl.ds(base + t * 16, 16)]
            pltpu.async_copy(x.at[wl96], rows_v, sem_io).wait()
            pltpu.async_copy(rows_v, nbx.at[rl96], sem_io).wait()
            return 0
        lax.fori_loop(0, nc, _scatter, 0)


def _sc_combine_kernel(hist, cc, out, pv, ccv, sem):
    wid = lax.axis_index("s") * _NC + lax.axis_index("c")
    col = wid * (_CP // _NW)

    def _row(r, acc):
        pltpu.sync_copy(hist.at[r, pl.ds(col, 32)], pv)
        return (acc[0] + pv[pl.ds(0, 16)], acc[1] + pv[pl.ds(16, 16)])
    acc = lax.fori_loop(
        0, _NW, _row,
        (jnp.zeros((16,), jnp.int32), jnp.zeros((16,), jnp.int32)))

    pltpu.sync_copy(cc.at[pl.ds(col, 32)], ccv)
    ccv[pl.ds(0, 16)] = ccv[pl.ds(0, 16)] + acc[0]
    ccv[pl.ds(16, 16)] = ccv[pl.ds(16, 16)] + acc[1]
    pltpu.sync_copy(ccv, out.at[pl.ds(col, 32)])


def kernel(bx, by, bu, class_counts, x, y, idx, uncertainty):
    mesh = plsc.VectorSubcoreMesh(core_axis_name="c", subcore_axis_name="s")

    update = pl.kernel(
        _sc_update_kernel,
        mesh=mesh,
        compiler_params=pltpu.CompilerParams(needs_layout_passes=False),
        out_type=[
            jax.ShapeDtypeStruct((_M, _D), jnp.float32),
            jax.ShapeDtypeStruct((_M,), jnp.int32),
            jax.ShapeDtypeStruct((_M,), jnp.float32),
            jax.ShapeDtypeStruct((_NW, _CP), jnp.int32),
        ],
        scratch_types=[
            pltpu.VMEM((_B,), jnp.int32),
            pltpu.VMEM((_B,), jnp.int32),
            pltpu.VMEM((_B,), jnp.float32),
            pltpu.VMEM((_R,), jnp.int32),
            pltpu.VMEM((_R,), jnp.float32),
            pltpu.VMEM((_R,), jnp.int32),
            pltpu.VMEM((_LISTPAD,), jnp.int32),
            pltpu.VMEM((_LISTPAD,), jnp.int32),
            pltpu.VMEM((_CHUNK,), jnp.int32),
            pltpu.VMEM((_CHUNK,), jnp.int32),
            pltpu.VMEM((_CHUNK, _D), jnp.float32),
            pltpu.VMEM((128,), jnp.int32),
            pltpu.VMEM((_CP,), jnp.int32),
            pltpu.SemaphoreType.DMA,
            pltpu.SemaphoreType.DMA,
        ],
    )
    new_bx, new_by, new_bu, hist = update(bx, by, bu, x, y, idx, uncertainty)

    combine = pl.kernel(
        _sc_combine_kernel,
        mesh=mesh,
        compiler_params=pltpu.CompilerParams(needs_layout_passes=False),
        out_type=jax.ShapeDtypeStruct((_CP,), jnp.int32),
        scratch_types=[
            pltpu.VMEM((32,), jnp.int32),
            pltpu.VMEM((32,), jnp.int32),
            pltpu.SemaphoreType.DMA,
        ],
    )
    cc_pad = jnp.pad(class_counts, (0, _CP - _C))
    new_cc = combine(hist, cc_pad)[: _C]

    return (new_bx, new_by, new_bu, new_cc)


# speedup vs baseline: 1.0005x; 1.0005x over previous
import functools

import jax
import jax.numpy as jnp
from jax import lax
from jax.experimental import pallas as pl
from jax.experimental.pallas import tpu as pltpu
from jax.experimental.pallas import tpu_sc as plsc

_M = 100000
_D = 512
_B = 16384
_C = 1000
_CP = 1024

_NC = 2
_NS = 16
_NW = _NC * _NS
_WCOPY = 25
_R = 4000
_BPW = _B // _NW
_CHUNK = 96
_NSPLIT = 5
_LISTPAD = _R + _CHUNK + 16

def _i16():
    return lax.broadcasted_iota(jnp.int32, (16,), 0)


def _take16(vec, ind):
    return lax.gather(
        vec, ind[:, None],
        lax.GatherDimensionNumbers(offset_dims=(), collapsed_slice_dims=(0,),
                                   start_index_map=(0,)),
        (1,), mode=lax.GatherScatterMode.PROMISE_IN_BOUNDS)


def _sc_update_kernel(bx, by, bu, x, y, idx, unc,
                      nbx, nby, nbu, hist,
                      idx_v, y_v, u_v, by_v, bu_v, win_v,
                      wl_v, rl_v, wl96, rl96, rows_v, lab_v, hist_v,
                      sem_copy, sem_io):
    wid = lax.axis_index("s") * _NC + lax.axis_index("c")
    lo = wid * _R

    is_range = wid < _WCOPY

    base = lo * jnp.where(is_range, 1, 0)
    copy_dmas = [
        pltpu.make_async_copy(
            bx.at[pl.ds(base + t * (_R // _NSPLIT), _R // _NSPLIT)],
            nbx.at[pl.ds(base + t * (_R // _NSPLIT), _R // _NSPLIT)],
            sem_copy.at[t])
        for t in range(_NSPLIT)
    ]

    @pl.when(is_range)
    def _():
        for d in copy_dmas:
            d.start()

    pltpu.sync_copy(idx, idx_v)
    pltpu.sync_copy(y, y_v)
    pltpu.sync_copy(unc, u_v)

    def _hist_zero(k, _):
        hist_v[pl.ds(k * 16, 16)] = jnp.zeros((16,), jnp.int32)
        return 0
    lax.fori_loop(0, _CP // 16, _hist_zero, 0, unroll=4)

    ibase = wid * _BPW
    nchunks = _BPW // 128
    for c in range(nchunks):
        off = ibase + c * 128
        pltpu.sync_copy(by.at[idx_v.at[pl.ds(off, 128)]], lab_v)

        def _hist_acc(k, _):
            mall = jnp.full((16,), True)
            lab = lab_v[pl.ds(k * 16, 16)]
            plsc.addupdate_scatter(hist_v, [lab],
                                   jnp.full((16,), -1, jnp.int32), mask=mall)
            yy = y_v[pl.ds(off + k * 16, 16)]
            plsc.addupdate_scatter(hist_v, [yy],
                                   jnp.ones((16,), jnp.int32), mask=mall)
            return 0
        lax.fori_loop(0, 8, _hist_acc, 0, unroll=2)

    pltpu.sync_copy(hist_v, hist.at[wid])

    @pl.when(is_range)
    def _():
        def _win_init(k, _):
            win_v[pl.ds(k * 16, 16)] = jnp.full((16,), -1, jnp.int32)
            return 0
        lax.fori_loop(0, _R // 16, _win_init, 0, unroll=4)

        shift_up = jnp.minimum(_i16() + 1, 15)

        def _win_build(k, _):
            v = idx_v[pl.ds(k * 16, 16)]
            rel = v - lo
            valid = (rel >= 0) & (rel < _R)
            ival = k * 16 + _i16()
            comp = jnp.where(valid, rel * _B + ival, jnp.int32(0x7FFFFFFF))
            scomp = lax.sort(comp)
            nxt = _take16(scomp, shift_up)
            srel = lax.shift_right_logical(scomp, 14)
            keep = ((srel != lax.shift_right_logical(nxt, 14)) | (_i16() == 15))
            keep = keep & (scomp != jnp.int32(0x7FFFFFFF))
            srel_c = jnp.where(keep, srel, 0)
            sival = jnp.where(keep, scomp & (_B - 1), 0)
            plsc.store_scatter(win_v, [srel_c], sival, mask=keep)
            return 0
        lax.fori_loop(0, _B // 16, _win_build, 0, unroll=2)

        pltpu.sync_copy(by.at[pl.ds(lo, _R)], by_v)
        pltpu.sync_copy(bu.at[pl.ds(lo, _R)], bu_v)

        def _merge(k, off):
            w = win_v[pl.ds(k * 16, 16)]
            m = w >= 0
            ws = jnp.where(m, w, 0)
            yw = plsc.load_gather(y_v, [ws], mask=m)
            uw = plsc.load_gather(u_v, [ws], mask=m)
            by_v[pl.ds(k * 16, 16)] = jnp.where(m, yw, by_v[pl.ds(k * 16, 16)])
            bu_v[pl.ds(k * 16, 16)] = jnp.where(m, uw, bu_v[pl.ds(k * 16, 16)])
            plsc.store_compressed(wl_v.at[pl.ds(off, 16)], ws, mask=m)
            plsc.store_compressed(rl_v.at[pl.ds(off, 16)],
                                  lo + k * 16 + _i16(), mask=m)
            return off + jnp.sum(m.astype(jnp.int32))
        nw = lax.fori_loop(0, _R // 16, _merge, jnp.int32(0))

        pltpu.sync_copy(by_v, nby.at[pl.ds(lo, _R)])
        pltpu.sync_copy(bu_v, nbu.at[pl.ds(lo, _R)])

        @pl.when(nw > 0)
        def _():
            lastw = wl_v[pl.ds(nw - 1, 16)]
            lastr = rl_v[pl.ds(nw - 1, 16)]
            z16 = jnp.zeros((16,), jnp.int32)
            padw = _take16(lastw, z16)
            padr = _take16(lastr, z16)
            for t in range(_CHUNK // 16):
                wl_v[pl.ds(nw + t * 16, 16)] = padw
                rl_v[pl.ds(nw + t * 16, 16)] = padr

        for d in copy_dmas:
            d.wait()

        nc = (nw + _CHUNK - 1) // _CHUNK

        def _scatter(c, _):
            base = c * _CHUNK
            for t in range(_CHUNK // 16):
                wl96[pl.ds(t * 16, 16)] = wl_v[pl.ds(base + t * 16, 16)]
                rl96[pl.ds(t * 16, 16)] = rl_v[pl.ds(base + t * 16, 16)]
            pltpu.async_copy(x.at[wl96], rows_v, sem_io).wait()
            pltpu.async_copy(rows_v, nbx.at[rl96], sem_io).wait()
            return 0
        lax.fori_loop(0, nc, _scatter, 0)


def _sc_combine_kernel(hist, cc, out, pv, ccv, sem):
    wid = lax.axis_index("s") * _NC + lax.axis_index("c")
    col = wid * (_CP // _NW)

    def _row(r, acc):
        pltpu.sync_copy(hist.at[r, pl.ds(col, 32)], pv)
        return (acc[0] + pv[pl.ds(0, 16)], acc[1] + pv[pl.ds(16, 16)])
    acc = lax.fori_loop(
        0, _NW, _row,
        (jnp.zeros((16,), jnp.int32), jnp.zeros((16,), jnp.int32)))

    pltpu.sync_copy(cc.at[pl.ds(col, 32)], ccv)
    ccv[pl.ds(0, 16)] = ccv[pl.ds(0, 16)] + acc[0]
    ccv[pl.ds(16, 16)] = ccv[pl.ds(16, 16)] + acc[1]
    pltpu.sync_copy(ccv, out.at[pl.ds(col, 32)])


def kernel(bx, by, bu, class_counts, x, y, idx, uncertainty):
    mesh = plsc.VectorSubcoreMesh(core_axis_name="c", subcore_axis_name="s")

    update = pl.kernel(
        _sc_update_kernel,
        mesh=mesh,
        compiler_params=pltpu.CompilerParams(needs_layout_passes=False),
        out_type=[
            jax.ShapeDtypeStruct((_M, _D), jnp.float32),
            jax.ShapeDtypeStruct((_M,), jnp.int32),
            jax.ShapeDtypeStruct((_M,), jnp.float32),
            jax.ShapeDtypeStruct((_NW, _CP), jnp.int32),
        ],
        scratch_types=[
            pltpu.VMEM((_B,), jnp.int32),
            pltpu.VMEM((_B,), jnp.int32),
            pltpu.VMEM((_B,), jnp.float32),
            pltpu.VMEM((_R,), jnp.int32),
            pltpu.VMEM((_R,), jnp.float32),
            pltpu.VMEM((_R,), jnp.int32),
            pltpu.VMEM((_LISTPAD,), jnp.int32),
            pltpu.VMEM((_LISTPAD,), jnp.int32),
            pltpu.VMEM((_CHUNK,), jnp.int32),
            pltpu.VMEM((_CHUNK,), jnp.int32),
            pltpu.VMEM((_CHUNK, _D), jnp.float32),
            pltpu.VMEM((128,), jnp.int32),
            pltpu.VMEM((_CP,), jnp.int32),
            pltpu.SemaphoreType.DMA((_NSPLIT,)),
            pltpu.SemaphoreType.DMA,
        ],
    )
    new_bx, new_by, new_bu, hist = update(bx, by, bu, x, y, idx, uncertainty)

    combine = pl.kernel(
        _sc_combine_kernel,
        mesh=mesh,
        compiler_params=pltpu.CompilerParams(needs_layout_passes=False),
        out_type=jax.ShapeDtypeStruct((_CP,), jnp.int32),
        scratch_types=[
            pltpu.VMEM((32,), jnp.int32),
            pltpu.VMEM((32,), jnp.int32),
            pltpu.SemaphoreType.DMA,
        ],
    )
    cc_pad = jnp.pad(class_counts, (0, _CP - _C))
    new_cc = combine(hist, cc_pad)[: _C]

    return (new_bx, new_by, new_bu, new_cc)


# speedup vs baseline: 18.0589x; 18.0501x over previous
import functools

import jax
import jax.numpy as jnp
from jax import lax
from jax.experimental import pallas as pl
from jax.experimental.pallas import tpu as pltpu
from jax.experimental.pallas import tpu_sc as plsc

_M = 100000
_D = 512
_B = 16384
_C = 1000
_CP = 1024

_NC = 2
_NS = 16
_NW = _NC * _NS
_WCOPY = 25
_R = 4000
_BPW = _B // _NW
_CHUNK = 96
_MBLK = 2000
_LISTPAD = _R + _CHUNK + 16

def _i16():
    return lax.broadcasted_iota(jnp.int32, (16,), 0)


def _take16(vec, ind):
    return lax.gather(
        vec, ind[:, None],
        lax.GatherDimensionNumbers(offset_dims=(), collapsed_slice_dims=(0,),
                                   start_index_map=(0,)),
        (1,), mode=lax.GatherScatterMode.PROMISE_IN_BOUNDS)


def _sc_update_kernel(by, bu, x, y, idx, unc,
                      nbx, nby, nbu, hist, win,
                      idx_v, y_v, u_v, by_v, bu_v, win_v,
                      wl_v, rl_v, wl96, rl96, rows_v, lab_v, hist_v,
                      sem_io):
    wid = lax.axis_index("s") * _NC + lax.axis_index("c")
    lo = wid * _R

    is_range = wid < _WCOPY

    pltpu.sync_copy(idx, idx_v)
    pltpu.sync_copy(y, y_v)
    pltpu.sync_copy(unc, u_v)

    def _hist_zero(k, _):
        hist_v[pl.ds(k * 16, 16)] = jnp.zeros((16,), jnp.int32)
        return 0
    lax.fori_loop(0, _CP // 16, _hist_zero, 0, unroll=4)

    ibase = wid * _BPW
    nchunks = _BPW // 128
    for c in range(nchunks):
        off = ibase + c * 128
        pltpu.sync_copy(by.at[idx_v.at[pl.ds(off, 128)]], lab_v)

        def _hist_acc(k, _):
            mall = jnp.full((16,), True)
            lab = lab_v[pl.ds(k * 16, 16)]
            plsc.addupdate_scatter(hist_v, [lab],
                                   jnp.full((16,), -1, jnp.int32), mask=mall)
            yy = y_v[pl.ds(off + k * 16, 16)]
            plsc.addupdate_scatter(hist_v, [yy],
                                   jnp.ones((16,), jnp.int32), mask=mall)
            return 0
        lax.fori_loop(0, 8, _hist_acc, 0, unroll=2)

    pltpu.sync_copy(hist_v, hist.at[wid])

    @pl.when(is_range)
    def _():
        def _win_init(k, _):
            win_v[pl.ds(k * 16, 16)] = jnp.full((16,), -1, jnp.int32)
            return 0
        lax.fori_loop(0, _R // 16, _win_init, 0, unroll=4)

        shift_up = jnp.minimum(_i16() + 1, 15)

        def _win_build(k, _):
            v = idx_v[pl.ds(k * 16, 16)]
            rel = v - lo
            valid = (rel >= 0) & (rel < _R)
            ival = k * 16 + _i16()
            comp = jnp.where(valid, rel * _B + ival, jnp.int32(0x7FFFFFFF))
            scomp = lax.sort(comp)
            nxt = _take16(scomp, shift_up)
            srel = lax.shift_right_logical(scomp, 14)
            keep = ((srel != lax.shift_right_logical(nxt, 14)) | (_i16() == 15))
            keep = keep & (scomp != jnp.int32(0x7FFFFFFF))
            srel_c = jnp.where(keep, srel, 0)
            sival = jnp.where(keep, scomp & (_B - 1), 0)
            plsc.store_scatter(win_v, [srel_c], sival, mask=keep)
            return 0
        lax.fori_loop(0, _B // 16, _win_build, 0, unroll=2)

        pltpu.sync_copy(by.at[pl.ds(lo, _R)], by_v)
        pltpu.sync_copy(bu.at[pl.ds(lo, _R)], bu_v)

        def _merge(k, off):
            w = win_v[pl.ds(k * 16, 16)]
            m = w >= 0
            ws = jnp.where(m, w, 0)
            yw = plsc.load_gather(y_v, [ws], mask=m)
            uw = plsc.load_gather(u_v, [ws], mask=m)
            by_v[pl.ds(k * 16, 16)] = jnp.where(m, yw, by_v[pl.ds(k * 16, 16)])
            bu_v[pl.ds(k * 16, 16)] = jnp.where(m, uw, bu_v[pl.ds(k * 16, 16)])
            plsc.store_compressed(wl_v.at[pl.ds(off, 16)], ws, mask=m)
            plsc.store_compressed(rl_v.at[pl.ds(off, 16)],
                                  lo + k * 16 + _i16(), mask=m)
            return off + jnp.sum(m.astype(jnp.int32))
        nw = lax.fori_loop(0, _R // 16, _merge, jnp.int32(0))

        pltpu.sync_copy(by_v, nby.at[pl.ds(lo, _R)])
        pltpu.sync_copy(bu_v, nbu.at[pl.ds(lo, _R)])
        pltpu.sync_copy(win_v, win.at[pl.ds(lo, _R)])

        @pl.when(nw > 0)
        def _():
            lastw = wl_v[pl.ds(nw - 1, 16)]
            lastr = rl_v[pl.ds(nw - 1, 16)]
            z16 = jnp.zeros((16,), jnp.int32)
            padw = _take16(lastw, z16)
            padr = _take16(lastr, z16)
            for t in range(_CHUNK // 16):
                wl_v[pl.ds(nw + t * 16, 16)] = padw
                rl_v[pl.ds(nw + t * 16, 16)] = padr

        nc = (nw + _CHUNK - 1) // _CHUNK

        def _scatter(c, _):
            base = c * _CHUNK
            for t in range(_CHUNK // 16):
                wl96[pl.ds(t * 16, 16)] = wl_v[pl.ds(base + t * 16, 16)]
                rl96[pl.ds(t * 16, 16)] = rl_v[pl.ds(base + t * 16, 16)]
            pltpu.async_copy(x.at[wl96], rows_v, sem_io).wait()
            pltpu.async_copy(rows_v, nbx.at[rl96], sem_io).wait()
            return 0
        lax.fori_loop(0, nc, _scatter, 0)


def _tc_merge_kernel(bx_ref, scat_ref, win_ref, out_ref):
    mask = win_ref[...] >= 0
    out_ref[...] = jnp.where(mask, scat_ref[...], bx_ref[...])


def _sc_combine_kernel(hist, cc, out, pv, ccv, sem):
    wid = lax.axis_index("s") * _NC + lax.axis_index("c")
    col = wid * (_CP // _NW)

    def _row(r, acc):
        pltpu.sync_copy(hist.at[r, pl.ds(col, 32)], pv)
        return (acc[0] + pv[pl.ds(0, 16)], acc[1] + pv[pl.ds(16, 16)])
    acc = lax.fori_loop(
        0, _NW, _row,
        (jnp.zeros((16,), jnp.int32), jnp.zeros((16,), jnp.int32)))

    pltpu.sync_copy(cc.at[pl.ds(col, 32)], ccv)
    ccv[pl.ds(0, 16)] = ccv[pl.ds(0, 16)] + acc[0]
    ccv[pl.ds(16, 16)] = ccv[pl.ds(16, 16)] + acc[1]
    pltpu.sync_copy(ccv, out.at[pl.ds(col, 32)])


def kernel(bx, by, bu, class_counts, x, y, idx, uncertainty):
    mesh = plsc.VectorSubcoreMesh(core_axis_name="c", subcore_axis_name="s")

    update = pl.kernel(
        _sc_update_kernel,
        mesh=mesh,
        compiler_params=pltpu.CompilerParams(needs_layout_passes=False),
        out_type=[
            jax.ShapeDtypeStruct((_M, _D), jnp.float32),
            jax.ShapeDtypeStruct((_M,), jnp.int32),
            jax.ShapeDtypeStruct((_M,), jnp.float32),
            jax.ShapeDtypeStruct((_NW, _CP), jnp.int32),
            jax.ShapeDtypeStruct((_M,), jnp.int32),
        ],
        scratch_types=[
            pltpu.VMEM((_B,), jnp.int32),
            pltpu.VMEM((_B,), jnp.int32),
            pltpu.VMEM((_B,), jnp.float32),
            pltpu.VMEM((_R,), jnp.int32),
            pltpu.VMEM((_R,), jnp.float32),
            pltpu.VMEM((_R,), jnp.int32),
            pltpu.VMEM((_LISTPAD,), jnp.int32),
            pltpu.VMEM((_LISTPAD,), jnp.int32),
            pltpu.VMEM((_CHUNK,), jnp.int32),
            pltpu.VMEM((_CHUNK,), jnp.int32),
            pltpu.VMEM((_CHUNK, _D), jnp.float32),
            pltpu.VMEM((128,), jnp.int32),
            pltpu.VMEM((_CP,), jnp.int32),
            pltpu.SemaphoreType.DMA,
        ],
    )
    bx_scat, new_by, new_bu, hist, win = update(by, bu, x, y, idx, uncertainty)

    merge_tc = pl.pallas_call(
        _tc_merge_kernel,
        grid=(_M // _MBLK,),
        in_specs=[
            pl.BlockSpec((_MBLK, _D), lambda i: (i, 0)),
            pl.BlockSpec((_MBLK, _D), lambda i: (i, 0)),
            pl.BlockSpec((_MBLK, 1), lambda i: (i, 0)),
        ],
        out_specs=pl.BlockSpec((_MBLK, _D), lambda i: (i, 0)),
        out_shape=jax.ShapeDtypeStruct((_M, _D), jnp.float32),
    )
    new_bx = merge_tc(bx, bx_scat, win.reshape(_M, 1))

    combine = pl.kernel(
        _sc_combine_kernel,
        mesh=mesh,
        compiler_params=pltpu.CompilerParams(needs_layout_passes=False),
        out_type=jax.ShapeDtypeStruct((_CP,), jnp.int32),
        scratch_types=[
            pltpu.VMEM((32,), jnp.int32),
            pltpu.VMEM((32,), jnp.int32),
            pltpu.SemaphoreType.DMA,
        ],
    )
    cc_pad = jnp.pad(class_counts, (0, _CP - _C))
    new_cc = combine(hist, cc_pad)[: _C]

    return (new_bx, new_by, new_bu, new_cc)
